# baseline (speedup 1.0000x reference)
import jax
import jax.numpy as jnp
from jax.experimental import pallas as pl
from jax.experimental.pallas import tpu as pltpu

B, C, L = 64, 384, 196
E, K = 8, 2
BOT = 96
LOSS_COEF = 0.01
G = 8
NSTEP = B // G
LCH = 28

_DN_CONTRACT1 = (((1,), (1,)), ((), ()))


def _gating_kernel(x_ref, wg_ref, w1_ref, w2_ref, w3_ref,
                   idx_ref, gv_ref, loss_ref, xb_ref,
                   w1b_ref, w2b_ref, w3b_ref, pooled_ref):
    i = pl.program_id(0)

    @pl.when(i == 0)
    def _():
        pooled_ref[...] = jnp.zeros_like(pooled_ref)
        w1b_ref[...] = w1_ref[...].astype(jnp.bfloat16)
        w2b_ref[...] = w2_ref[...].astype(jnp.bfloat16)
        w3b_ref[...] = w3_ref[...].astype(jnp.bfloat16)

    chunk = x_ref[...]
    xb_ref[...] = chunk.astype(jnp.bfloat16)
    pooled_ref[...] += jnp.sum(chunk, axis=0)

    @pl.when(i == pl.num_programs(0) - 1)
    def _():
        pooled = pooled_ref[...] * (1.0 / L)
        logits = jax.lax.dot_general(pooled, wg_ref[...], _DN_CONTRACT1,
                                     preferred_element_type=jnp.float32)
        mx = jnp.max(logits, axis=1, keepdims=True)
        ex = jnp.exp(logits - mx)
        probs = ex / jnp.sum(ex, axis=1, keepdims=True)

        iota = jax.lax.broadcasted_iota(jnp.int32, (B, E), 1)
        m1 = jnp.max(probs, axis=1, keepdims=True)
        a1 = jnp.min(jnp.where(probs == m1, iota, E), axis=1)
        masked = jnp.where(iota == a1[:, None], -1.0, probs)
        m2 = jnp.max(masked, axis=1, keepdims=True)
        a2 = jnp.min(jnp.where(masked == m2, iota, E), axis=1)

        e2 = jnp.exp(m2 - m1)
        g1 = 1.0 / (1.0 + e2)
        g2 = e2 * g1

        oh1 = (iota == a1[:, None]).astype(jnp.float32)
        oh2 = (iota == a2[:, None]).astype(jnp.float32)
        importance = jnp.sum(oh1 * g1 + oh2 * g2, axis=0)
        load = jnp.sum(oh1 + oh2, axis=0)

        def cv2(v):
            mean = jnp.mean(v)
            var = jnp.sum((v - mean) ** 2) / (E - 1)
            return var / (mean * mean + 1e-10)

        loss = LOSS_COEF * (cv2(importance) + cv2(load))
        loss_ref[...] = jnp.reshape(loss, (1, 1))
        idx_ref[...] = jnp.stack([a1, a2], axis=1).astype(jnp.int32)
        gv_ref[...] = jnp.concatenate([g1, g2], axis=1)


def _expert_chain(xb_bf, w1, w2, w3, b1v, b2v, b3v):
    h = jax.lax.dot_general(xb_bf, w1, _DN_CONTRACT1,
                            preferred_element_type=jnp.float32)
    h = jnp.maximum(h + b1v[None, :], 0.0).astype(jnp.bfloat16)
    zero = jnp.zeros((1, BOT), dtype=jnp.bfloat16)
    hp = jnp.concatenate([zero, h, zero], axis=0)
    acc = jax.lax.dot_general(hp[0:L], w2[0], _DN_CONTRACT1,
                              preferred_element_type=jnp.float32)
    acc += jax.lax.dot_general(hp[1:L + 1], w2[1], _DN_CONTRACT1,
                               preferred_element_type=jnp.float32)
    acc += jax.lax.dot_general(hp[2:L + 2], w2[2], _DN_CONTRACT1,
                               preferred_element_type=jnp.float32)
    h2 = jnp.maximum(acc + b2v[None, :], 0.0).astype(jnp.bfloat16)
    y = jnp.dot(h2, w3, preferred_element_type=jnp.float32)
    return y + b3v[None, :]


def _dispatch_kernel(idx_ref, gv_ref, x_ref, w1_ref, w2_ref, w3_ref,
                     b1_ref, b2_ref, b3_ref, y_hbm,
                     ybuf, out_sem):
    i = pl.program_id(0)
    s = i % 2

    @pl.when(i >= 2)
    def _():
        for j in range(G):
            pltpu.make_async_copy(ybuf.at[s, j],
                                  y_hbm.at[:, (i - 2) * G + j, :],
                                  out_sem.at[s, j]).wait()

    for j in range(G):
        b = i * G + j
        xb_bf = x_ref[:, j, :]
        acc = None
        for k in range(K):
            e = idx_ref[b, k]
            g = gv_ref[b, k]
            y = _expert_chain(xb_bf, w1_ref[e], w2_ref[e], w3_ref[e],
                              b1_ref[e], b2_ref[e], b3_ref[e])
            term = g * jnp.maximum(y + xb_bf.astype(jnp.float32), 0.0)
            acc = term if acc is None else acc + term
        ybuf[s, j] = acc
        pltpu.make_async_copy(ybuf.at[s, j], y_hbm.at[:, b, :],
                              out_sem.at[s, j]).start()

    @pl.when(i == NSTEP - 1)
    def _():
        for st in (NSTEP - 2, NSTEP - 1):
            for j in range(G):
                pltpu.make_async_copy(ybuf.at[st % 2, j],
                                      y_hbm.at[:, st * G + j, :],
                                      out_sem.at[st % 2, j]).wait()


@jax.jit
def kernel(x, w_gate, w1, b1, w2, b2, w3, b3):
    xt = jnp.transpose(x, (2, 0, 1))
    wgt = jnp.transpose(w_gate, (1, 0))
    w2t = jnp.transpose(w2, (0, 3, 1, 2))
    w1f = w1.reshape(E, BOT, C)
    w3f = jnp.transpose(w3, (0, 2, 3, 1)).reshape(E, BOT, C)

    idx, gv, loss, xbf, w1b, w2b, w3b = pl.pallas_call(
        _gating_kernel,
        grid=(L // LCH,),
        in_specs=[
            pl.BlockSpec((LCH, B, C), lambda i: (i, 0, 0)),
            pl.BlockSpec((E, C), lambda i: (0, 0)),
            pl.BlockSpec((E, BOT, C), lambda i: (0, 0, 0)),
            pl.BlockSpec((E, 3, BOT, BOT), lambda i: (0, 0, 0, 0)),
            pl.BlockSpec((E, BOT, C), lambda i: (0, 0, 0)),
        ],
        out_specs=(
            pl.BlockSpec((B, K), lambda i: (0, 0)),
            pl.BlockSpec((B, K), lambda i: (0, 0)),
            pl.BlockSpec((1, 1), lambda i: (0, 0)),
            pl.BlockSpec((LCH, B, C), lambda i: (i, 0, 0)),
            pl.BlockSpec((E, BOT, C), lambda i: (0, 0, 0)),
            pl.BlockSpec((E, 3, BOT, BOT), lambda i: (0, 0, 0, 0)),
            pl.BlockSpec((E, BOT, C), lambda i: (0, 0, 0)),
        ),
        out_shape=(
            jax.ShapeDtypeStruct((B, K), jnp.int32),
            jax.ShapeDtypeStruct((B, K), jnp.float32),
            jax.ShapeDtypeStruct((1, 1), jnp.float32),
            jax.ShapeDtypeStruct((L, B, C), jnp.bfloat16),
            jax.ShapeDtypeStruct((E, BOT, C), jnp.bfloat16),
            jax.ShapeDtypeStruct((E, 3, BOT, BOT), jnp.bfloat16),
            jax.ShapeDtypeStruct((E, BOT, C), jnp.bfloat16),
        ),
        scratch_shapes=[pltpu.VMEM((B, C), jnp.float32)],
    )(xt, wgt, w1f, w2t, w3f)

    grid_spec = pltpu.PrefetchScalarGridSpec(
        num_scalar_prefetch=2,
        grid=(NSTEP,),
        in_specs=[
            pl.BlockSpec((L, G, C), lambda i, idx, gv: (0, i, 0)),
            pl.BlockSpec((E, BOT, C), lambda i, idx, gv: (0, 0, 0)),
            pl.BlockSpec((E, 3, BOT, BOT), lambda i, idx, gv: (0, 0, 0, 0)),
            pl.BlockSpec((E, BOT, C), lambda i, idx, gv: (0, 0, 0)),
            pl.BlockSpec((E, BOT), lambda i, idx, gv: (0, 0)),
            pl.BlockSpec((E, BOT), lambda i, idx, gv: (0, 0)),
            pl.BlockSpec((E, C), lambda i, idx, gv: (0, 0)),
        ],
        out_specs=pl.BlockSpec(memory_space=pl.ANY),
        scratch_shapes=[
            pltpu.VMEM((2, G, L, C), jnp.float32),
            pltpu.SemaphoreType.DMA((2, G)),
        ],
    )
    yt = pl.pallas_call(
        _dispatch_kernel,
        grid_spec=grid_spec,
        out_shape=jax.ShapeDtypeStruct((L, B, C), jnp.float32),
    )(idx, gv, xbf, w1b, w2b, w3b, b1, b2, b3)

    y = jnp.transpose(yt, (1, 2, 0))
    return (y, loss.reshape(()))


# speedup vs baseline: 1.0282x; 1.0282x over previous
import jax
import jax.numpy as jnp
from jax.experimental import pallas as pl
from jax.experimental.pallas import tpu as pltpu

B, C, L = 64, 384, 196
E, K = 8, 2
BOT = 96
LOSS_COEF = 0.01
G = 8
NSTEP = B // G
LCH = 28

_DN_CONTRACT1 = (((1,), (1,)), ((), ()))


def _gating_kernel(x_ref, wg_ref, w1_ref, w2_ref, w3_ref,
                   idx_ref, gv_ref, loss_ref,
                   w1b_ref, w2b_ref, w3b_ref, pooled_ref):
    i = pl.program_id(0)

    @pl.when(i == 0)
    def _():
        pooled_ref[...] = jnp.zeros_like(pooled_ref)
        w1b_ref[...] = w1_ref[...].astype(jnp.bfloat16)
        w2b_ref[...] = w2_ref[...].astype(jnp.bfloat16)
        w3b_ref[...] = w3_ref[...].astype(jnp.bfloat16)

    pooled_ref[...] += jnp.sum(x_ref[...], axis=0)

    @pl.when(i == pl.num_programs(0) - 1)
    def _():
        pooled = pooled_ref[...] * (1.0 / L)
        logits = jax.lax.dot_general(pooled, wg_ref[...], _DN_CONTRACT1,
                                     preferred_element_type=jnp.float32)
        mx = jnp.max(logits, axis=1, keepdims=True)
        ex = jnp.exp(logits - mx)
        probs = ex / jnp.sum(ex, axis=1, keepdims=True)

        iota = jax.lax.broadcasted_iota(jnp.int32, (B, E), 1)
        m1 = jnp.max(probs, axis=1, keepdims=True)
        a1 = jnp.min(jnp.where(probs == m1, iota, E), axis=1)
        masked = jnp.where(iota == a1[:, None], -1.0, probs)
        m2 = jnp.max(masked, axis=1, keepdims=True)
        a2 = jnp.min(jnp.where(masked == m2, iota, E), axis=1)

        e2 = jnp.exp(m2 - m1)
        g1 = 1.0 / (1.0 + e2)
        g2 = e2 * g1

        oh1 = (iota == a1[:, None]).astype(jnp.float32)
        oh2 = (iota == a2[:, None]).astype(jnp.float32)
        importance = jnp.sum(oh1 * g1 + oh2 * g2, axis=0)
        load = jnp.sum(oh1 + oh2, axis=0)

        def cv2(v):
            mean = jnp.mean(v)
            var = jnp.sum((v - mean) ** 2) / (E - 1)
            return var / (mean * mean + 1e-10)

        loss = LOSS_COEF * (cv2(importance) + cv2(load))
        loss_ref[...] = jnp.reshape(loss, (1, 1))
        idx_ref[...] = jnp.stack([a1, a2], axis=1).astype(jnp.int32)
        gv_ref[...] = jnp.concatenate([g1, g2], axis=1)


def _expert_chain(xb_bf, w1, w2, w3, b1v, b2v, b3v):
    h = jax.lax.dot_general(xb_bf, w1, _DN_CONTRACT1,
                            preferred_element_type=jnp.float32)
    h = jnp.maximum(h + b1v[None, :], 0.0).astype(jnp.bfloat16)
    zero = jnp.zeros((1, BOT), dtype=jnp.bfloat16)
    hp = jnp.concatenate([zero, h, zero], axis=0)
    acc = jax.lax.dot_general(hp[0:L], w2[0], _DN_CONTRACT1,
                              preferred_element_type=jnp.float32)
    acc += jax.lax.dot_general(hp[1:L + 1], w2[1], _DN_CONTRACT1,
                               preferred_element_type=jnp.float32)
    acc += jax.lax.dot_general(hp[2:L + 2], w2[2], _DN_CONTRACT1,
                               preferred_element_type=jnp.float32)
    h2 = jnp.maximum(acc + b2v[None, :], 0.0).astype(jnp.bfloat16)
    y = jnp.dot(h2, w3, preferred_element_type=jnp.float32)
    return y + b3v[None, :]


def _dispatch_kernel(idx_ref, gv_ref, x_ref, w1_ref, w2_ref, w3_ref,
                     b1_ref, b2_ref, b3_ref, y_hbm,
                     ybuf, out_sem):
    i = pl.program_id(0)
    s = i % 2

    @pl.when(i >= 2)
    def _():
        for j in range(G):
            pltpu.make_async_copy(ybuf.at[s, j],
                                  y_hbm.at[:, (i - 2) * G + j, :],
                                  out_sem.at[s, j]).wait()

    for j in range(G):
        b = i * G + j
        xb = x_ref[:, j, :]
        xb_bf = xb.astype(jnp.bfloat16)
        acc = None
        for k in range(K):
            e = idx_ref[b, k]
            g = gv_ref[b, k]
            y = _expert_chain(xb_bf, w1_ref[e], w2_ref[e], w3_ref[e],
                              b1_ref[e], b2_ref[e], b3_ref[e])
            term = g * jnp.maximum(y + xb, 0.0)
            acc = term if acc is None else acc + term
        ybuf[s, j] = acc
        pltpu.make_async_copy(ybuf.at[s, j], y_hbm.at[:, b, :],
                              out_sem.at[s, j]).start()

    @pl.when(i == NSTEP - 1)
    def _():
        for st in (NSTEP - 2, NSTEP - 1):
            for j in range(G):
                pltpu.make_async_copy(ybuf.at[st % 2, j],
                                      y_hbm.at[:, st * G + j, :],
                                      out_sem.at[st % 2, j]).wait()


@jax.jit
def kernel(x, w_gate, w1, b1, w2, b2, w3, b3):
    xt = jnp.transpose(x, (2, 0, 1))
    wgt = jnp.transpose(w_gate, (1, 0))
    w2t = jnp.transpose(w2, (0, 3, 1, 2))
    w1f = w1.reshape(E, BOT, C)
    w3f = jnp.transpose(w3, (0, 2, 3, 1)).reshape(E, BOT, C)

    idx, gv, loss, w1b, w2b, w3b = pl.pallas_call(
        _gating_kernel,
        grid=(L // LCH,),
        in_specs=[
            pl.BlockSpec((LCH, B, C), lambda i: (i, 0, 0)),
            pl.BlockSpec((E, C), lambda i: (0, 0)),
            pl.BlockSpec((E, BOT, C), lambda i: (0, 0, 0)),
            pl.BlockSpec((E, 3, BOT, BOT), lambda i: (0, 0, 0, 0)),
            pl.BlockSpec((E, BOT, C), lambda i: (0, 0, 0)),
        ],
        out_specs=(
            pl.BlockSpec((B, K), lambda i: (0, 0)),
            pl.BlockSpec((B, K), lambda i: (0, 0)),
            pl.BlockSpec((1, 1), lambda i: (0, 0)),
            pl.BlockSpec((E, BOT, C), lambda i: (0, 0, 0)),
            pl.BlockSpec((E, 3, BOT, BOT), lambda i: (0, 0, 0, 0)),
            pl.BlockSpec((E, BOT, C), lambda i: (0, 0, 0)),
        ),
        out_shape=(
            jax.ShapeDtypeStruct((B, K), jnp.int32),
            jax.ShapeDtypeStruct((B, K), jnp.float32),
            jax.ShapeDtypeStruct((1, 1), jnp.float32),
            jax.ShapeDtypeStruct((E, BOT, C), jnp.bfloat16),
            jax.ShapeDtypeStruct((E, 3, BOT, BOT), jnp.bfloat16),
            jax.ShapeDtypeStruct((E, BOT, C), jnp.bfloat16),
        ),
        scratch_shapes=[pltpu.VMEM((B, C), jnp.float32)],
    )(xt, wgt, w1f, w2t, w3f)

    grid_spec = pltpu.PrefetchScalarGridSpec(
        num_scalar_prefetch=2,
        grid=(NSTEP,),
        in_specs=[
            pl.BlockSpec((L, G, C), lambda i, idx, gv: (0, i, 0)),
            pl.BlockSpec((E, BOT, C), lambda i, idx, gv: (0, 0, 0)),
            pl.BlockSpec((E, 3, BOT, BOT), lambda i, idx, gv: (0, 0, 0, 0)),
            pl.BlockSpec((E, BOT, C), lambda i, idx, gv: (0, 0, 0)),
            pl.BlockSpec((E, BOT), lambda i, idx, gv: (0, 0)),
            pl.BlockSpec((E, BOT), lambda i, idx, gv: (0, 0)),
            pl.BlockSpec((E, C), lambda i, idx, gv: (0, 0)),
        ],
        out_specs=pl.BlockSpec(memory_space=pl.ANY),
        scratch_shapes=[
            pltpu.VMEM((2, G, L, C), jnp.float32),
            pltpu.SemaphoreType.DMA((2, G)),
        ],
    )
    yt = pl.pallas_call(
        _dispatch_kernel,
        grid_spec=grid_spec,
        out_shape=jax.ShapeDtypeStruct((L, B, C), jnp.float32),
    )(idx, gv, xt, w1b, w2b, w3b, b1, b2, b3)

    y = jnp.transpose(yt, (1, 2, 0))
    return (y, loss.reshape(()))
